# baseline (speedup 1.0000x reference)
import jax
import jax.numpy as jnp
from jax.experimental import pallas as pl

_N = 50000
_F = 3
_T = 32
_S = 32
_NUM_SEGMENTS = 128
_SCALE = 500.0

_BLK = 512


def _ect_body(x_ref, b_ref, v_ref, lin_ref, out_ref):
    i = pl.program_id(0)

    @pl.when(i == 0)
    def _init():
        out_ref[:, :] = jnp.zeros_like(out_ref)

    xb = x_ref[:, :]
    nh = jnp.dot(xb, v_ref[:, :], preferred_element_type=jnp.float32)
    ecc = jax.nn.sigmoid(_SCALE * (lin_ref[0, :][None, :] - nh))
    bids = b_ref[0, 0, :]
    rows = jax.lax.broadcasted_iota(jnp.int32, (_NUM_SEGMENTS, _BLK), 0)
    onehot = jnp.where(rows == bids[None, :], 1.0, 0.0).astype(jnp.bfloat16)
    out_ref[:, :] += jnp.dot(onehot, ecc.astype(jnp.bfloat16),
                             preferred_element_type=jnp.float32)


def kernel(x, batch, v, lin):
    n = x.shape[0]
    nb = (n + _BLK - 1) // _BLK
    npad = nb * _BLK
    x_p = jnp.pad(x, ((0, npad - n), (0, 0)))
    b_p = jnp.pad(batch, (0, npad - n), constant_values=_NUM_SEGMENTS)
    b_p = b_p.reshape(nb, 1, _BLK)
    v_flat = jnp.tile(v, (1, _S))
    lin_flat = jnp.repeat(lin, _T).reshape(1, _S * _T)

    out = pl.pallas_call(
        _ect_body,
        grid=(nb,),
        in_specs=[
            pl.BlockSpec((_BLK, _F), lambda i: (i, 0)),
            pl.BlockSpec((1, 1, _BLK), lambda i: (i, 0, 0)),
            pl.BlockSpec((_F, _S * _T), lambda i: (0, 0)),
            pl.BlockSpec((1, _S * _T), lambda i: (0, 0)),
        ],
        out_specs=pl.BlockSpec((_NUM_SEGMENTS, _S * _T), lambda i: (0, 0)),
        out_shape=jax.ShapeDtypeStruct((_NUM_SEGMENTS, _S * _T), jnp.float32),
    )(x_p, b_p, v_flat, lin_flat)
    return out.reshape(_NUM_SEGMENTS, _S, _T)


# speedup vs baseline: 1.2903x; 1.2903x over previous
import jax
import jax.numpy as jnp
from jax.experimental import pallas as pl

_N = 50000
_F = 3
_T = 32
_S = 32
_NUM_SEGMENTS = 128
_SCALE = 500.0

_BLK = 1000


def _ect_body(x_ref, b_ref, v_ref, lin_ref, out_ref):
    i = pl.program_id(0)

    @pl.when(i == 0)
    def _init():
        out_ref[:, :] = jnp.zeros_like(out_ref)

    xb = x_ref[:, :]
    nh = jnp.dot(xb, v_ref[:, :], preferred_element_type=jnp.float32)
    ecc = jax.nn.sigmoid(_SCALE * (lin_ref[0, :][None, :] - nh))
    bids = b_ref[0, 0, :]
    rows = jax.lax.broadcasted_iota(jnp.int32, (_NUM_SEGMENTS, _BLK), 0)
    onehot = jnp.where(rows == bids[None, :], 1.0, 0.0).astype(jnp.bfloat16)
    out_ref[:, :] += jnp.dot(onehot, ecc.astype(jnp.bfloat16),
                             preferred_element_type=jnp.float32)


def kernel(x, batch, v, lin):
    n = x.shape[0]
    nb = n // _BLK
    b_r = batch.reshape(nb, 1, _BLK)
    v_flat = jnp.tile(v, (1, _S))
    lin_flat = jnp.repeat(lin, _T).reshape(1, _S * _T)

    out = pl.pallas_call(
        _ect_body,
        grid=(nb,),
        in_specs=[
            pl.BlockSpec((_BLK, _F), lambda i: (i, 0)),
            pl.BlockSpec((1, 1, _BLK), lambda i: (i, 0, 0)),
            pl.BlockSpec((_F, _S * _T), lambda i: (0, 0)),
            pl.BlockSpec((1, _S * _T), lambda i: (0, 0)),
        ],
        out_specs=pl.BlockSpec((_NUM_SEGMENTS, _S * _T), lambda i: (0, 0)),
        out_shape=jax.ShapeDtypeStruct((_NUM_SEGMENTS, _S * _T), jnp.float32),
    )(x, b_r, v_flat, lin_flat)
    return out.reshape(_NUM_SEGMENTS, _S, _T)
